# baseline (speedup 1.0000x reference)
import jax
import jax.numpy as jnp
from jax import lax
from jax.experimental import pallas as pl
from jax.experimental.pallas import tpu as pltpu
from jax.experimental.pallas import tpu_sc as plsc

N = 10000
D = 128
H = 128
C = 40
E = 320000

NC = 2
NS = 16
NW = NC * NS
L = 16
CHUNK = 128
CHA = 63
CHB = 95
CHT = CHA + CHB
EPT = CHT * CHUNK
E_PAD = NS * EPT
PAD = E_PAD - E
N_PAD = 10112
NCH = N_PAD // CHUNK
NJR = -(-NCH // NS)

_f32 = jnp.float32


def _sc_agg_builder(with_cnt: bool):
  mesh = plsc.VectorSubcoreMesh(core_axis_name="c", subcore_axis_name="s")
  out_type = [jax.ShapeDtypeStruct((NC, N_PAD, H), _f32)]
  scratch = [
      pltpu.VMEM((CHUNK,), jnp.int32),
      pltpu.VMEM((CHUNK,), jnp.int32),
      pltpu.VMEM((CHUNK, H), _f32),
      pltpu.VMEM_SHARED((N_PAD, H), _f32),
      pltpu.SemaphoreType.DMA,
  ]
  if with_cnt:
    out_type.append(jax.ShapeDtypeStruct((NC, NS, N_PAD), _f32))
    scratch.append(pltpu.VMEM((N_PAD,), _f32))

  def body(table, srcp2, dstp2, zrows, *rest):
    if with_cnt:
      (out_acc, out_cnt, sidx, didx, rowsa, acc, sema, cnt_v) = rest
    else:
      (out_acc, sidx, didx, rowsa, acc, sema) = rest
    c = lax.axis_index("c")
    t = lax.axis_index("s")
    wid = t * NC + c

    pltpu.sync_copy(zrows, rowsa)

    def zstep(j, carry):
      k = t + NS * j
      @pl.when(k < NCH)
      def _():
        pltpu.sync_copy(rowsa, acc.at[pl.ds(k * CHUNK, CHUNK)])
      return carry

    lax.fori_loop(0, NJR, zstep, 0)
    if with_cnt:
      zv = jnp.zeros((L,), _f32)

      def czstep(i, carry):
        cnt_v[pl.ds(i * L, L)] = zv
        return carry

      lax.fori_loop(0, N_PAD // L, czstep, 0)
    plsc.subcore_barrier()

    onev = jnp.ones((L,), _f32)
    mk = jnp.ones((L,), jnp.bool_)
    base = t * EPT + jnp.where(c == 0, 0, CHA * CHUNK)
    my_ch = jnp.where(c == 0, CHA, CHB)

    def step(i, carry):
      @pl.when(i < my_ch)
      def _():
        off = base + i * CHUNK
        pltpu.sync_copy(srcp2.at[pl.ds(off, CHUNK)], sidx)
        g = pltpu.async_copy(table.at[sidx], rowsa, sema)
        pltpu.sync_copy(dstp2.at[pl.ds(off, CHUNK)], didx)
        if with_cnt:
          for l in range(CHUNK // L):
            idxv = didx[pl.ds(l * L, L)]
            plsc.addupdate_scatter(cnt_v, [idxv], onev, mask=mk)
        g.wait()
        pltpu.sync_copy(rowsa, acc.at[didx], add=True)
      return carry

    lax.fori_loop(0, max(CHA, CHB), step, 0)
    plsc.subcore_barrier()

    def ostep(j, carry):
      k = t + NS * j
      @pl.when(k < NCH)
      def _():
        sl = pl.ds(k * CHUNK, CHUNK)
        pltpu.sync_copy(acc.at[sl], rowsa)
        pltpu.sync_copy(rowsa, out_acc.at[c, sl])
      return carry

    lax.fori_loop(0, NJR, ostep, 0)
    if with_cnt:
      pltpu.sync_copy(cnt_v, out_cnt.at[c, t])

  return pl.kernel(
      body, out_type=out_type, mesh=mesh, scratch_types=scratch,
      compiler_params=pltpu.CompilerParams(needs_layout_passes=False))


_sc_agg_cnt = _sc_agg_builder(True)
_sc_agg = _sc_agg_builder(False)


RB = 1000


def _tc1_body(x_ref, wl_ref, wr_ref, b_ref, y_ref, z_ref):
  xb = x_ref[...]
  y_ref[...] = jnp.dot(xb, wl_ref[...], preferred_element_type=_f32)
  z_ref[...] = jnp.dot(xb, wr_ref[...], preferred_element_type=_f32) + b_ref[...]


def _tc2_body(aggp_ref, cntp_ref, z1_ref, wl_ref, wr_ref, b_ref,
              y2_ref, z2_ref, inv_ref):
  agg = aggp_ref[0] + aggp_ref[1]
  cnt = jnp.sum(cntp_ref[...], axis=1, keepdims=True)
  inv = 1.0 / jnp.maximum(cnt, 1.0)
  h = jnp.maximum(agg * inv + z1_ref[...], 0.0)
  y2_ref[...] = jnp.dot(h, wl_ref[...], preferred_element_type=_f32)
  z2_ref[...] = jnp.dot(h, wr_ref[...], preferred_element_type=_f32) + b_ref[...]
  inv_ref[...] = jnp.broadcast_to(inv, (RB, H))


def _tc3_body(aggp_ref, inv_ref, z2_ref, wc_ref, b_ref, o_ref):
  agg = aggp_ref[0] + aggp_ref[1]
  h2 = agg * inv_ref[:, 0:1] + z2_ref[...]
  o_ref[...] = jnp.dot(h2, wc_ref[...], preferred_element_type=_f32) + b_ref[...]


def _row_spec(shape):
  if len(shape) == 2:
    return pl.BlockSpec((RB, shape[1]), lambda i: (i, 0))
  return pl.BlockSpec((shape[0], RB, shape[2]), lambda i: (0, i, 0))


def _full_spec(shape):
  return pl.BlockSpec(shape, lambda i: tuple(0 for _ in shape))


def _tc_call(body, ins, outs):
  grid = (N // RB,)
  in_specs = []
  for a in ins:
    if a.shape[0] == N or (a.ndim == 3 and a.shape[1] == N):
      in_specs.append(_row_spec(a.shape))
    else:
      in_specs.append(_full_spec(a.shape))
  out_shape = [jax.ShapeDtypeStruct(s, _f32) for s in outs]
  out_specs = [pl.BlockSpec((RB, s[1]), lambda i: (i, 0)) for s in outs]
  return pl.pallas_call(
      body, grid=grid, in_specs=in_specs, out_specs=out_specs,
      out_shape=out_shape)(*ins)


def kernel(x, edge_index, W1l, b1, W1r, W2l, b2, W2r, Wc, bc):
  src = edge_index[0]
  dst = edge_index[1]
  srcp = jnp.concatenate([src, jnp.zeros((PAD,), jnp.int32)])
  dstp = jnp.concatenate([dst, jnp.full((PAD,), N, jnp.int32)])
  zrows = jnp.zeros((CHUNK, H), _f32)

  y1, z1 = _tc_call(_tc1_body, (x, W1l, W1r, b1.reshape(1, H)),
                    [(N, H), (N, H)])
  agg1p, cntp = _sc_agg_cnt(y1, srcp, dstp, zrows)
  cntp2 = cntp.reshape(NC * NS, N_PAD)[:, :N].T
  y2, z2, invc = _tc_call(
      _tc2_body, (agg1p[:, :N], cntp2, z1, W2l, W2r,
                  b2.reshape(1, H)),
      [(N, H), (N, H), (N, H)])
  (agg2p,) = _sc_agg(y2, srcp, dstp, zrows)
  (out,) = _tc_call(_tc3_body, (agg2p[:, :N], invc, z2, Wc, bc.reshape(1, C)),
                    [(N, C)])
  return out


# speedup vs baseline: 1.1717x; 1.1717x over previous
import jax
import jax.numpy as jnp
from jax import lax
from jax.experimental import pallas as pl
from jax.experimental.pallas import tpu as pltpu
from jax.experimental.pallas import tpu_sc as plsc

N = 10000
D = 128
H = 128
C = 40
E = 320000

NC = 2
NS = 16
NW = NC * NS
L = 16
CHUNK = 128
CHA = 95
CHB = 63
CHT = CHA + CHB
EPT = CHT * CHUNK
E_PAD = NS * EPT
PAD = E_PAD - E
N_PAD = 10112
NCH = N_PAD // CHUNK
NJR = -(-NCH // NS)

_f32 = jnp.float32


def _sc_agg_builder(with_cnt: bool):
  mesh = plsc.VectorSubcoreMesh(core_axis_name="c", subcore_axis_name="s")
  out_type = [jax.ShapeDtypeStruct((NC, N_PAD, H), _f32)]
  scratch = [
      pltpu.VMEM((CHUNK,), jnp.int32),
      pltpu.VMEM((CHUNK,), jnp.int32),
      pltpu.VMEM((CHUNK, H), _f32),
      pltpu.VMEM_SHARED((N_PAD, H), _f32),
      pltpu.SemaphoreType.DMA,
  ]
  if with_cnt:
    out_type.append(jax.ShapeDtypeStruct((NC, NS, N_PAD), _f32))
    scratch.append(pltpu.VMEM((N_PAD,), _f32))

  def body(table, srcp2, dstp2, zrows, *rest):
    if with_cnt:
      (out_acc, out_cnt, sidx, didx, rowsa, acc, sema, cnt_v) = rest
    else:
      (out_acc, sidx, didx, rowsa, acc, sema) = rest
    c = lax.axis_index("c")
    t = lax.axis_index("s")
    wid = t * NC + c

    pltpu.sync_copy(zrows, rowsa)

    def zstep(j, carry):
      k = t + NS * j
      @pl.when(k < NCH)
      def _():
        pltpu.sync_copy(rowsa, acc.at[pl.ds(k * CHUNK, CHUNK)])
      return carry

    lax.fori_loop(0, NJR, zstep, 0)
    if with_cnt:
      zv = jnp.zeros((L,), _f32)

      def czstep(i, carry):
        cnt_v[pl.ds(i * L, L)] = zv
        return carry

      lax.fori_loop(0, N_PAD // L, czstep, 0)
    plsc.subcore_barrier()

    onev = jnp.ones((L,), _f32)
    mk = jnp.ones((L,), jnp.bool_)
    base = t * EPT + jnp.where(c == 0, 0, CHA * CHUNK)
    my_ch = jnp.where(c == 0, CHA, CHB)

    def step(i, carry):
      @pl.when(i < my_ch)
      def _():
        off = base + i * CHUNK
        pltpu.sync_copy(srcp2.at[pl.ds(off, CHUNK)], sidx)
        g = pltpu.async_copy(table.at[sidx], rowsa, sema)
        pltpu.sync_copy(dstp2.at[pl.ds(off, CHUNK)], didx)
        if with_cnt:
          for l in range(CHUNK // L):
            idxv = didx[pl.ds(l * L, L)]
            plsc.addupdate_scatter(cnt_v, [idxv], onev, mask=mk)
        g.wait()
        pltpu.sync_copy(rowsa, acc.at[didx], add=True)
      return carry

    lax.fori_loop(0, max(CHA, CHB), step, 0)
    plsc.subcore_barrier()

    def ostep(j, carry):
      k = t + NS * j
      @pl.when(k < NCH)
      def _():
        sl = pl.ds(k * CHUNK, CHUNK)
        pltpu.sync_copy(acc.at[sl], rowsa)
        pltpu.sync_copy(rowsa, out_acc.at[c, sl])
      return carry

    lax.fori_loop(0, NJR, ostep, 0)
    if with_cnt:
      pltpu.sync_copy(cnt_v, out_cnt.at[c, t])

  return pl.kernel(
      body, out_type=out_type, mesh=mesh, scratch_types=scratch,
      compiler_params=pltpu.CompilerParams(needs_layout_passes=False))


_sc_agg_cnt = _sc_agg_builder(True)
_sc_agg = _sc_agg_builder(False)


RB = 1000


def _tc1_body(x_ref, wl_ref, wr_ref, b_ref, y_ref, z_ref):
  xb = x_ref[...]
  y_ref[...] = jnp.dot(xb, wl_ref[...], preferred_element_type=_f32)
  z_ref[...] = jnp.dot(xb, wr_ref[...], preferred_element_type=_f32) + b_ref[...]


def _tc2_body(aggp_ref, cntp_ref, z1_ref, wl_ref, wr_ref, b_ref,
              y2_ref, z2_ref, inv_ref):
  agg = aggp_ref[0] + aggp_ref[1]
  cnt = jnp.sum(cntp_ref[...], axis=1, keepdims=True)
  inv = 1.0 / jnp.maximum(cnt, 1.0)
  h = jnp.maximum(agg * inv + z1_ref[...], 0.0)
  y2_ref[...] = jnp.dot(h, wl_ref[...], preferred_element_type=_f32)
  z2_ref[...] = jnp.dot(h, wr_ref[...], preferred_element_type=_f32) + b_ref[...]
  inv_ref[...] = jnp.broadcast_to(inv, (RB, H))


def _tc3_body(aggp_ref, inv_ref, z2_ref, wc_ref, b_ref, o_ref):
  agg = aggp_ref[0] + aggp_ref[1]
  h2 = agg * inv_ref[:, 0:1] + z2_ref[...]
  o_ref[...] = jnp.dot(h2, wc_ref[...], preferred_element_type=_f32) + b_ref[...]


def _row_spec(shape):
  if len(shape) == 2:
    return pl.BlockSpec((RB, shape[1]), lambda i: (i, 0))
  return pl.BlockSpec((shape[0], RB, shape[2]), lambda i: (0, i, 0))


def _full_spec(shape):
  return pl.BlockSpec(shape, lambda i: tuple(0 for _ in shape))


def _tc_call(body, ins, outs):
  grid = (N // RB,)
  in_specs = []
  for a in ins:
    if a.shape[0] == N or (a.ndim == 3 and a.shape[1] == N):
      in_specs.append(_row_spec(a.shape))
    else:
      in_specs.append(_full_spec(a.shape))
  out_shape = [jax.ShapeDtypeStruct(s, _f32) for s in outs]
  out_specs = [pl.BlockSpec((RB, s[1]), lambda i: (i, 0)) for s in outs]
  return pl.pallas_call(
      body, grid=grid, in_specs=in_specs, out_specs=out_specs,
      out_shape=out_shape)(*ins)


def kernel(x, edge_index, W1l, b1, W1r, W2l, b2, W2r, Wc, bc):
  src = edge_index[0]
  dst = edge_index[1]
  srcp = jnp.concatenate([src, jnp.zeros((PAD,), jnp.int32)])
  dstp = jnp.concatenate([dst, jnp.full((PAD,), N, jnp.int32)])
  zrows = jnp.zeros((CHUNK, H), _f32)

  y1, z1 = _tc_call(_tc1_body, (x, W1l, W1r, b1.reshape(1, H)),
                    [(N, H), (N, H)])
  agg1p, cntp = _sc_agg_cnt(y1, srcp, dstp, zrows)
  cntp2 = cntp.reshape(NC * NS, N_PAD)[:, :N].T
  y2, z2, invc = _tc_call(
      _tc2_body, (agg1p[:, :N], cntp2, z1, W2l, W2r,
                  b2.reshape(1, H)),
      [(N, H), (N, H), (N, H)])
  (agg2p,) = _sc_agg(y2, srcp, dstp, zrows)
  (out,) = _tc_call(_tc3_body, (agg2p[:, :N], invc, z2, Wc, bc.reshape(1, C)),
                    [(N, C)])
  return out
